# Initial kernel scaffold; baseline (speedup 1.0000x reference)
import jax
import jax.numpy as jnp
from jax.experimental import pallas as pl
from jax.experimental.pallas import tpu as pltpu

B, T, D = 1, 2048, 768
FF = 1536
E = 8
N = B * T
BT = 256

_HI = jax.lax.Precision.HIGHEST


def _silu(v):
    return v * jax.nn.sigmoid(v)


def _dot(a, b):
    return jax.lax.dot_general(a, b, (((1,), (0,)), ((), ())),
                               preferred_element_type=jnp.float32)


def _router_shared_kernel(x_ref, rw_ref, gu_ref, dw_ref, sg_ref,
                          vals_ref, idx_ref, sh_ref):
    x = x_ref[...]
    logits = jax.lax.dot_general(x, rw_ref[...], (((1,), (0,)), ((), ())),
                                 precision=_HI,
                                 preferred_element_type=jnp.float32)
    m = jnp.max(logits, axis=-1, keepdims=True)
    p = jnp.exp(logits - m)
    p = p / jnp.sum(p, axis=-1, keepdims=True)
    iota = jax.lax.broadcasted_iota(jnp.int32, p.shape, 1)
    m1 = jnp.max(p, axis=-1, keepdims=True)
    i1 = jnp.min(jnp.where(p == m1, iota, E), axis=-1, keepdims=True)
    pm = jnp.where(iota == i1, -jnp.inf, p)
    m2 = jnp.max(pm, axis=-1, keepdims=True)
    i2 = jnp.min(jnp.where(pm == m2, iota, E), axis=-1, keepdims=True)
    vals_ref[...] = jnp.concatenate([m1, m2], axis=1)
    idx_ref[...] = jnp.concatenate([i1, i2], axis=1)
    xb = x.astype(jnp.bfloat16)
    gu = _dot(xb, gu_ref[...])
    h = (_silu(gu[:, :FF]) * gu[:, FF:]).astype(jnp.bfloat16)
    sh = _dot(h, dw_ref[...])
    sgl = jax.lax.dot_general(x, sg_ref[...], (((1,), (0,)), ((), ())),
                              precision=_HI,
                              preferred_element_type=jnp.float32)
    sh_ref[...] = sh * jax.nn.sigmoid(sgl)


def _experts_kernel(xb_ref, vals_ref, idx_ref, sh_ref, w1_ref, w2_ref,
                    out_ref):
    e = pl.program_id(0)
    xb = xb_ref[...]
    h = _dot(xb, w1_ref[0])
    hb = _silu(h).astype(jnp.bfloat16)
    o = _dot(hb, w2_ref[0])
    w = jnp.sum(jnp.where(idx_ref[...] == e, vals_ref[...], 0.0),
                axis=1, keepdims=True)

    @pl.when(e == 0)
    def _():
        out_ref[...] = sh_ref[...]

    out_ref[...] += w * o


def kernel(x, router_w, w1, w2, gate_up_w, down_w, shared_gate_w):
    Bv, Tv, Dv = x.shape
    flat = x.reshape(N, D)

    vals, idx, sh = pl.pallas_call(
        _router_shared_kernel,
        grid=(N // BT,),
        in_specs=[
            pl.BlockSpec((BT, D), lambda t: (t, 0)),
            pl.BlockSpec((D, E), lambda t: (0, 0)),
            pl.BlockSpec((D, 2 * FF), lambda t: (0, 0)),
            pl.BlockSpec((FF, D), lambda t: (0, 0)),
            pl.BlockSpec((D, 1), lambda t: (0, 0)),
        ],
        out_specs=[
            pl.BlockSpec((BT, 2), lambda t: (t, 0)),
            pl.BlockSpec((BT, 2), lambda t: (t, 0)),
            pl.BlockSpec((BT, D), lambda t: (t, 0)),
        ],
        out_shape=[
            jax.ShapeDtypeStruct((N, 2), jnp.float32),
            jax.ShapeDtypeStruct((N, 2), jnp.int32),
            jax.ShapeDtypeStruct((N, D), jnp.float32),
        ],
    )(flat, router_w, gate_up_w.astype(jnp.bfloat16),
      down_w.astype(jnp.bfloat16), shared_gate_w)

    out = pl.pallas_call(
        _experts_kernel,
        grid=(E,),
        in_specs=[
            pl.BlockSpec((N, D), lambda e: (0, 0)),
            pl.BlockSpec((N, 2), lambda e: (0, 0)),
            pl.BlockSpec((N, 2), lambda e: (0, 0)),
            pl.BlockSpec((N, D), lambda e: (0, 0)),
            pl.BlockSpec((1, D, FF), lambda e: (e, 0, 0)),
            pl.BlockSpec((1, FF, D), lambda e: (e, 0, 0)),
        ],
        out_specs=pl.BlockSpec((N, D), lambda e: (0, 0)),
        out_shape=jax.ShapeDtypeStruct((N, D), jnp.float32),
        compiler_params=pltpu.CompilerParams(
            dimension_semantics=("arbitrary",)),
    )(flat.astype(jnp.bfloat16), vals, idx, sh,
      w1.astype(jnp.bfloat16), w2.astype(jnp.bfloat16))

    return out.reshape(Bv, Tv, Dv)


# speedup vs baseline: 1.5364x; 1.5364x over previous
import jax
import jax.numpy as jnp
from jax.experimental import pallas as pl
from jax.experimental.pallas import tpu as pltpu

B, T, D = 1, 2048, 768
FF = 1536
E = 8
N = B * T
BT = 256

_HI = jax.lax.Precision.HIGHEST


def _silu(v):
    return v * jax.nn.sigmoid(v)


def _dot(a, b):
    return jax.lax.dot_general(a, b, (((1,), (0,)), ((), ())),
                               preferred_element_type=jnp.float32)


def _router_shared_kernel(x_ref, rw_ref, gu_ref, dw_ref, sg_ref,
                          vals_ref, idx_ref, sh_ref):
    x = x_ref[...]
    xb = x.astype(jnp.bfloat16)
    logits = _dot(xb, rw_ref[...])
    m = jnp.max(logits, axis=-1, keepdims=True)
    p = jnp.exp(logits - m)
    p = p / jnp.sum(p, axis=-1, keepdims=True)
    iota = jax.lax.broadcasted_iota(jnp.int32, p.shape, 1)
    m1 = jnp.max(p, axis=-1, keepdims=True)
    i1 = jnp.min(jnp.where(p == m1, iota, E), axis=-1, keepdims=True)
    pm = jnp.where(iota == i1, -jnp.inf, p)
    m2 = jnp.max(pm, axis=-1, keepdims=True)
    i2 = jnp.min(jnp.where(pm == m2, iota, E), axis=-1, keepdims=True)
    vals_ref[...] = jnp.concatenate([m1, m2], axis=1)
    idx_ref[...] = jnp.concatenate([i1, i2], axis=1)
    gu = _dot(xb, gu_ref[...])
    h = (_silu(gu[:, :FF]) * gu[:, FF:]).astype(jnp.bfloat16)
    sh = _dot(h, dw_ref[...])
    sgl = _dot(xb, sg_ref[...])
    sh_ref[...] = sh * jax.nn.sigmoid(sgl)


def _experts_kernel(xb_ref, vals_ref, idx_ref, sh_ref, w1_ref, w2_ref,
                    out_ref):
    e = pl.program_id(0)
    xb = xb_ref[...]
    h = _dot(xb, w1_ref[0])
    hb = _silu(h).astype(jnp.bfloat16)
    o = _dot(hb, w2_ref[0])
    w = jnp.sum(jnp.where(idx_ref[...] == e, vals_ref[...], 0.0),
                axis=1, keepdims=True)

    @pl.when(e == 0)
    def _():
        out_ref[...] = sh_ref[...]

    out_ref[...] += w * o


def kernel(x, router_w, w1, w2, gate_up_w, down_w, shared_gate_w):
    Bv, Tv, Dv = x.shape
    flat = x.reshape(N, D)

    vals, idx, sh = pl.pallas_call(
        _router_shared_kernel,
        grid=(N // BT,),
        in_specs=[
            pl.BlockSpec((BT, D), lambda t: (t, 0)),
            pl.BlockSpec((D, E), lambda t: (0, 0)),
            pl.BlockSpec((D, 2 * FF), lambda t: (0, 0)),
            pl.BlockSpec((FF, D), lambda t: (0, 0)),
            pl.BlockSpec((D, 1), lambda t: (0, 0)),
        ],
        out_specs=[
            pl.BlockSpec((BT, 2), lambda t: (t, 0)),
            pl.BlockSpec((BT, 2), lambda t: (t, 0)),
            pl.BlockSpec((BT, D), lambda t: (t, 0)),
        ],
        out_shape=[
            jax.ShapeDtypeStruct((N, 2), jnp.float32),
            jax.ShapeDtypeStruct((N, 2), jnp.int32),
            jax.ShapeDtypeStruct((N, D), jnp.float32),
        ],
    )(flat, router_w.astype(jnp.bfloat16), gate_up_w.astype(jnp.bfloat16),
      down_w.astype(jnp.bfloat16), shared_gate_w.astype(jnp.bfloat16))

    out = pl.pallas_call(
        _experts_kernel,
        grid=(E,),
        in_specs=[
            pl.BlockSpec((N, D), lambda e: (0, 0)),
            pl.BlockSpec((N, 2), lambda e: (0, 0)),
            pl.BlockSpec((N, 2), lambda e: (0, 0)),
            pl.BlockSpec((N, D), lambda e: (0, 0)),
            pl.BlockSpec((1, D, FF), lambda e: (e, 0, 0)),
            pl.BlockSpec((1, FF, D), lambda e: (e, 0, 0)),
        ],
        out_specs=pl.BlockSpec((N, D), lambda e: (0, 0)),
        out_shape=jax.ShapeDtypeStruct((N, D), jnp.float32),
        compiler_params=pltpu.CompilerParams(
            dimension_semantics=("arbitrary",)),
    )(flat.astype(jnp.bfloat16), vals, idx, sh,
      w1.astype(jnp.bfloat16), w2.astype(jnp.bfloat16))

    return out.reshape(Bv, Tv, Dv)
